# baseline (speedup 1.0000x reference)
import functools

import jax
import jax.numpy as jnp
from jax.experimental import pallas as pl
from jax.experimental.pallas import tpu as pltpu


M_TILE = 512


def _mlp_kernel(x_ref, w1_ref, b1_ref, w2_ref, b2_ref, out_ref, w1b_ref,
                ha_ref, hb_ref):
    i = pl.program_id(0)
    n = pl.num_programs(0)

    @pl.when(i == 0)
    def _():
        w1b_ref[...] = w1_ref[...].astype(jnp.bfloat16)

    def stage(cur_ref, prev_ref):
        @pl.when(i < n - 1)
        def _():
            x = x_ref[...].astype(jnp.bfloat16)
            cur_ref[...] = jnp.dot(x, w1b_ref[...],
                                   preferred_element_type=jnp.float32)

        @pl.when(i > 0)
        def _():
            h = jnp.maximum(prev_ref[...] + b1_ref[...], 0.0)
            logits = jnp.sum(h * w2_ref[...], axis=1) + b2_ref[0, 0]
            out_ref[0, 0, :] = jax.nn.sigmoid(logits)

    @pl.when(i % 2 == 0)
    def _():
        stage(ha_ref, hb_ref)

    @pl.when(i % 2 == 1)
    def _():
        stage(hb_ref, ha_ref)


@functools.partial(jax.jit, static_argnames=())
def kernel(img_features, W1, b1, W2, b2):
    n, d = img_features.shape
    num_tiles = n // M_TILE
    b1r = b1.reshape(1, d)
    w2r = W2.reshape(1, d)
    b2r = b2.reshape(1, 1)
    out = pl.pallas_call(
        _mlp_kernel,
        grid=(num_tiles + 1,),
        in_specs=[
            pl.BlockSpec((M_TILE, d), lambda i: (jnp.minimum(i, num_tiles - 1), 0)),
            pl.BlockSpec((d, d), lambda i: (0, 0)),
            pl.BlockSpec((1, d), lambda i: (0, 0)),
            pl.BlockSpec((1, d), lambda i: (0, 0)),
            pl.BlockSpec((1, 1), lambda i: (0, 0)),
        ],
        out_specs=pl.BlockSpec((1, 1, M_TILE),
                               lambda i: (jnp.maximum(i - 1, 0), 0, 0)),
        out_shape=jax.ShapeDtypeStruct((num_tiles, 1, M_TILE), jnp.float32),
        scratch_shapes=[
            pltpu.VMEM((d, d), jnp.bfloat16),
            pltpu.VMEM((M_TILE, d), jnp.float32),
            pltpu.VMEM((M_TILE, d), jnp.float32),
        ],
    )(img_features, W1, b1r, w2r, b2r)
    return out.reshape(n, 1)


# speedup vs baseline: 1.0918x; 1.0918x over previous
import functools

import jax
import jax.numpy as jnp
from jax.experimental import pallas as pl
from jax.experimental.pallas import tpu as pltpu


M_TILE = 512


def _mlp_kernel(x_ref, w1_ref, b1_ref, w2_ref, b2_ref, out_ref, w1b_ref):
    @pl.when(pl.program_id(0) == 0)
    def _():
        w1b_ref[...] = w1_ref[...].astype(jnp.bfloat16)

    x = x_ref[...].astype(jnp.bfloat16)
    h = jnp.dot(x, w1b_ref[...], preferred_element_type=jnp.float32)
    h = jnp.maximum(h + b1_ref[...], 0.0)
    logits = jnp.sum(h * w2_ref[...], axis=1) + b2_ref[0, 0]
    out_ref[0, 0, :] = 0.5 * jnp.tanh(0.5 * logits) + 0.5


@functools.partial(jax.jit, static_argnames=())
def kernel(img_features, W1, b1, W2, b2):
    n, d = img_features.shape
    num_tiles = n // M_TILE
    b1r = b1.reshape(1, d)
    w2r = W2.reshape(1, d)
    b2r = b2.reshape(1, 1)
    out = pl.pallas_call(
        _mlp_kernel,
        grid=(num_tiles,),
        in_specs=[
            pl.BlockSpec((M_TILE, d), lambda i: (i, 0)),
            pl.BlockSpec((d, d), lambda i: (0, 0)),
            pl.BlockSpec((1, d), lambda i: (0, 0)),
            pl.BlockSpec((1, d), lambda i: (0, 0)),
            pl.BlockSpec((1, 1), lambda i: (0, 0)),
        ],
        out_specs=pl.BlockSpec((1, 1, M_TILE), lambda i: (i, 0, 0)),
        out_shape=jax.ShapeDtypeStruct((num_tiles, 1, M_TILE), jnp.float32),
        scratch_shapes=[pltpu.VMEM((d, d), jnp.bfloat16)],
    )(img_features, W1, b1r, w2r, b2r)
    return out.reshape(n, 1)


# speedup vs baseline: 1.1456x; 1.0493x over previous
import functools

import jax
import jax.numpy as jnp
from jax.experimental import pallas as pl
from jax.experimental.pallas import tpu as pltpu


M_TILE = 1024
M_SUB = 512


def _mlp_kernel(x_ref, w1_ref, b1_ref, w2_ref, b2_ref, out_ref, w1b_ref):
    @pl.when(pl.program_id(0) == 0)
    def _():
        w1b_ref[...] = w1_ref[...].astype(jnp.bfloat16)

    n_sub = M_TILE // M_SUB
    for s in range(n_sub):
        x = x_ref[pl.ds(s * M_SUB, M_SUB), :].astype(jnp.bfloat16)
        h = jnp.dot(x, w1b_ref[...], preferred_element_type=jnp.float32)
        h = jnp.maximum(h + b1_ref[...], 0.0)
        logits = jnp.sum(h * w2_ref[...], axis=1) + b2_ref[0, 0]
        out_ref[0, 0, pl.ds(s * M_SUB, M_SUB)] = 0.5 * jnp.tanh(0.5 * logits) + 0.5


@functools.partial(jax.jit, static_argnames=())
def kernel(img_features, W1, b1, W2, b2):
    n, d = img_features.shape
    num_tiles = n // M_TILE
    b1r = b1.reshape(1, d)
    w2r = W2.reshape(1, d)
    b2r = b2.reshape(1, 1)
    out = pl.pallas_call(
        _mlp_kernel,
        grid=(num_tiles,),
        in_specs=[
            pl.BlockSpec((M_TILE, d), lambda i: (i, 0)),
            pl.BlockSpec((d, d), lambda i: (0, 0)),
            pl.BlockSpec((1, d), lambda i: (0, 0)),
            pl.BlockSpec((1, d), lambda i: (0, 0)),
            pl.BlockSpec((1, 1), lambda i: (0, 0)),
        ],
        out_specs=pl.BlockSpec((1, 1, M_TILE), lambda i: (i, 0, 0)),
        out_shape=jax.ShapeDtypeStruct((num_tiles, 1, M_TILE), jnp.float32),
        scratch_shapes=[pltpu.VMEM((d, d), jnp.bfloat16)],
    )(img_features, W1, b1r, w2r, b2r)
    return out.reshape(n, 1)
